# baseline (speedup 1.0000x reference)
import functools

import jax
import jax.numpy as jnp
import numpy as np
from jax import lax
from jax.experimental import pallas as pl
from jax.experimental.pallas import tpu as pltpu
from jax.experimental.pallas import tpu_sc as plsc

B, C, H, W = 8, 96, 224, 224
N = H * W
BINS = 10
NBIN = N // BINS
NBINP = 5024
NB_BLOCKS = 1
NBLOCK = N // NB_BLOCKS
NTILES = 4
PER_TILE = N // NTILES
RED_CH = NBINP // 16


def _tc_body(x_ref, t_ref, nll_ref, bin_ref):
    xb = x_ref[0]
    t = t_ref[0]
    m = jnp.max(xb, axis=0, keepdims=True)
    s = jnp.sum(jnp.exp(xb - m), axis=0, keepdims=True)
    cls = lax.broadcasted_iota(jnp.int32, (C, NBLOCK), 0)
    xt = jnp.sum(jnp.where(cls == t, xb, 0.0), axis=0, keepdims=True)
    logp = xt - m - jnp.log(s)
    nll_ref[0] = -logp
    g = jnp.abs(jnp.exp(logp) - 1.0)
    bf = jnp.floor(g * np.float32(NBIN - 0.0001))
    bin_ref[0] = jnp.minimum(bf, NBIN - 1).astype(jnp.int32)


_tc_stats = pl.pallas_call(
    _tc_body,
    grid=(B * NB_BLOCKS,),
    in_specs=[
        pl.BlockSpec((1, C, NBLOCK), lambda i: (i // NB_BLOCKS, 0, i % NB_BLOCKS)),
        pl.BlockSpec((1, 1, NBLOCK), lambda i: (i // NB_BLOCKS, 0, i % NB_BLOCKS)),
    ],
    out_specs=[
        pl.BlockSpec((1, 1, NBLOCK), lambda i: (i // NB_BLOCKS, 0, i % NB_BLOCKS)),
        pl.BlockSpec((1, 1, NBLOCK), lambda i: (i // NB_BLOCKS, 0, i % NB_BLOCKS)),
    ],
    out_shape=[
        jax.ShapeDtypeStruct((B, 1, N), jnp.float32),
        jax.ShapeDtypeStruct((B, 1, N), jnp.int32),
    ],
)


@functools.cache
def _make_sc_kernel():
    mesh = plsc.VectorSubcoreMesh(core_axis_name="c", subcore_axis_name="s")

    @functools.partial(
        pl.kernel,
        out_type=jax.ShapeDtypeStruct((B, 16), jnp.float32),
        mesh=mesh,
        compiler_params=pltpu.CompilerParams(
            needs_layout_passes=False, use_tc_tiling_on_sc=False
        ),
        scratch_types=[
            pltpu.VMEM((16 * NBINP,), jnp.float32),
            pltpu.VMEM((PER_TILE,), jnp.int32),
            pltpu.VMEM((PER_TILE,), jnp.float32),
            pltpu.VMEM((NBINP,), jnp.float32),
            pltpu.VMEM((NBINP,), jnp.float32),
            pltpu.VMEM((NBINP,), jnp.float32),
            pltpu.VMEM((16,), jnp.float32),
            pltpu.VMEM_SHARED((16, 2, NBINP), jnp.float32),
        ],
    )
    def sc_hist(bins_hbm, nll_hbm, out_hbm, repl, binsv, nllv, cpart, spart,
                tmp, outv, shared):
        cid = lax.axis_index("c")
        sid = lax.axis_index("s")
        b = cid * (16 // NTILES) + sid // NTILES
        member = sid % NTILES
        off = b * N + member * PER_TILE

        zero16 = jnp.zeros((16,), jnp.float32)
        ones16 = jnp.full((16,), 1.0, jnp.float32)
        lane = lax.broadcasted_iota(jnp.int32, (16,), 0)

        def zero_body(j, _):
            repl[pl.ds(j * 16, 16)] = zero16
            return 0

        lax.fori_loop(0, 16 * NBINP // 16, zero_body, 0)

        pltpu.sync_copy(bins_hbm.at[pl.ds(off, PER_TILE)], binsv)
        pltpu.sync_copy(nll_hbm.at[pl.ds(off, PER_TILE)], nllv)

        def scat_c(i, _):
            idx = binsv[pl.ds(i * 16, 16)]
            plsc.addupdate_scatter(repl, [lane * NBINP + idx], ones16)
            return 0

        lax.fori_loop(0, PER_TILE // 16, scat_c, 0)

        def red_c(k, _):
            base = k * 16
            acc = repl[pl.ds(base, 16)]
            repl[pl.ds(base, 16)] = zero16
            for l in range(1, 16):
                o = l * NBINP + base
                acc = acc + repl[pl.ds(o, 16)]
                repl[pl.ds(o, 16)] = zero16
            cpart[pl.ds(base, 16)] = acc
            return 0

        lax.fori_loop(0, RED_CH, red_c, 0)

        def scat_s(i, _):
            idx = binsv[pl.ds(i * 16, 16)]
            vals = nllv[pl.ds(i * 16, 16)]
            plsc.addupdate_scatter(repl, [lane * NBINP + idx], vals)
            return 0

        lax.fori_loop(0, PER_TILE // 16, scat_s, 0)

        def red_s(k, _):
            base = k * 16
            acc = repl[pl.ds(base, 16)]
            for l in range(1, 16):
                acc = acc + repl[pl.ds(l * NBINP + base, 16)]
            spart[pl.ds(base, 16)] = acc
            return 0

        lax.fori_loop(0, RED_CH, red_s, 0)

        pltpu.sync_copy(cpart, shared.at[sid, 0])
        pltpu.sync_copy(spart, shared.at[sid, 1])
        plsc.subcore_barrier()

        base_slot = sid - member
        for d in range(1, NTILES):
            peer = base_slot + (member + d) % NTILES
            pltpu.sync_copy(shared.at[peer, 0], tmp)

            def addc(k, _):
                sl = pl.ds(k * 16, 16)
                cpart[sl] = cpart[sl] + tmp[sl]
                return 0

            lax.fori_loop(0, RED_CH, addc, 0)
            pltpu.sync_copy(shared.at[peer, 1], tmp)

            def adds(k, _):
                sl = pl.ds(k * 16, 16)
                spart[sl] = spart[sl] + tmp[sl]
                return 0

            lax.fori_loop(0, RED_CH, adds, 0)

        @pl.when(member == 0)
        def _():
            def fin(k, carry):
                ne_a, t_a = carry
                sl = pl.ds(k * 16, 16)
                cc = cpart[sl]
                ss = spart[sl]
                ne_a = ne_a + jnp.where(cc > 0.0, 1.0, 0.0)
                t_a = t_a + ss / jnp.maximum(cc, 1.0)
                return ne_a, t_a

            ne16, term16 = lax.fori_loop(0, RED_CH, fin, (zero16, zero16))
            term_v = zero16 + jnp.sum(term16)
            ne_v = zero16 + jnp.sum(ne16)
            outv[...] = term_v / ne_v
            pltpu.sync_copy(outv, out_hbm.at[b])

    return sc_hist


def kernel(x, target):
    x3 = x.reshape(B, C, N)
    t3 = target.reshape(B, 1, N)
    nll3, bin3 = _tc_stats(x3, t3)
    per_sample = _make_sc_kernel()(bin3.reshape(-1), nll3.reshape(-1))
    return jnp.mean(per_sample[:, 0])


# speedup vs baseline: 1.0833x; 1.0833x over previous
import functools

import jax
import jax.numpy as jnp
import numpy as np
from jax import lax
from jax.experimental import pallas as pl
from jax.experimental.pallas import tpu as pltpu
from jax.experimental.pallas import tpu_sc as plsc

B, C, H, W = 8, 96, 224, 224
N = H * W
BINS = 10
NBIN = N // BINS
NBINP = 5024
NB_BLOCKS = 2
NBLOCK = N // NB_BLOCKS
NTILES = 4
PER_TILE = N // NTILES
RED_CH = NBINP // 16


def _tc_body(x_ref, t_ref, nll_ref, bin_ref):
    xb = x_ref[0]
    t = t_ref[0]
    m = jnp.max(xb, axis=0, keepdims=True)
    s = jnp.sum(jnp.exp(xb - m), axis=0, keepdims=True)
    cls = lax.broadcasted_iota(jnp.int32, (C, NBLOCK), 0)
    xt = jnp.sum(jnp.where(cls == t, xb, 0.0), axis=0, keepdims=True)
    logp = xt - m - jnp.log(s)
    nll_ref[0] = -logp
    g = jnp.abs(jnp.exp(logp) - 1.0)
    bf = jnp.floor(g * np.float32(NBIN - 0.0001))
    bin_ref[0] = jnp.minimum(bf, NBIN - 1).astype(jnp.int32)


_tc_stats = pl.pallas_call(
    _tc_body,
    grid=(B * NB_BLOCKS,),
    in_specs=[
        pl.BlockSpec((1, C, NBLOCK), lambda i: (i // NB_BLOCKS, 0, i % NB_BLOCKS)),
        pl.BlockSpec((1, 1, NBLOCK), lambda i: (i // NB_BLOCKS, 0, i % NB_BLOCKS)),
    ],
    out_specs=[
        pl.BlockSpec((1, 1, NBLOCK), lambda i: (i // NB_BLOCKS, 0, i % NB_BLOCKS)),
        pl.BlockSpec((1, 1, NBLOCK), lambda i: (i // NB_BLOCKS, 0, i % NB_BLOCKS)),
    ],
    out_shape=[
        jax.ShapeDtypeStruct((B, 1, N), jnp.float32),
        jax.ShapeDtypeStruct((B, 1, N), jnp.int32),
    ],
)


@functools.cache
def _make_sc_kernel():
    mesh = plsc.VectorSubcoreMesh(core_axis_name="c", subcore_axis_name="s")

    @functools.partial(
        pl.kernel,
        out_type=jax.ShapeDtypeStruct((B, 16), jnp.float32),
        mesh=mesh,
        compiler_params=pltpu.CompilerParams(
            needs_layout_passes=False, use_tc_tiling_on_sc=False
        ),
        scratch_types=[
            pltpu.VMEM((16 * NBINP,), jnp.float32),
            pltpu.VMEM((PER_TILE,), jnp.int32),
            pltpu.VMEM((PER_TILE,), jnp.float32),
            pltpu.VMEM((NBINP,), jnp.float32),
            pltpu.VMEM((NBINP,), jnp.float32),
            pltpu.VMEM((NBINP,), jnp.float32),
            pltpu.VMEM((16,), jnp.float32),
            pltpu.VMEM_SHARED((16, 2, NBINP), jnp.float32),
        ],
    )
    def sc_hist(bins_hbm, nll_hbm, out_hbm, repl, binsv, nllv, cpart, spart,
                tmp, outv, shared):
        cid = lax.axis_index("c")
        sid = lax.axis_index("s")
        b = cid * (16 // NTILES) + sid // NTILES
        member = sid % NTILES
        off = b * N + member * PER_TILE

        zero16 = jnp.zeros((16,), jnp.float32)
        ones16 = jnp.full((16,), 1.0, jnp.float32)
        lane = lax.broadcasted_iota(jnp.int32, (16,), 0)

        def zero_body(j, _):
            for u in range(8):
                repl[pl.ds(j * 128 + u * 16, 16)] = zero16
            return 0

        lax.fori_loop(0, 16 * NBINP // 128, zero_body, 0)

        pltpu.sync_copy(bins_hbm.at[pl.ds(off, PER_TILE)], binsv)
        pltpu.sync_copy(nll_hbm.at[pl.ds(off, PER_TILE)], nllv)

        lane_off = lane * NBINP

        def scat_c(i, _):
            for u in range(2):
                idx = binsv[pl.ds(i * 32 + u * 16, 16)]
                plsc.addupdate_scatter(repl, [lane_off + idx], ones16)
            return 0

        lax.fori_loop(0, PER_TILE // 32, scat_c, 0)

        def red_c(k, _):
            base = k * 16
            acc = repl[pl.ds(base, 16)]
            repl[pl.ds(base, 16)] = zero16
            for l in range(1, 16):
                o = l * NBINP + base
                acc = acc + repl[pl.ds(o, 16)]
                repl[pl.ds(o, 16)] = zero16
            cpart[pl.ds(base, 16)] = acc
            return 0

        lax.fori_loop(0, RED_CH, red_c, 0)

        def scat_s(i, _):
            for u in range(2):
                idx = binsv[pl.ds(i * 32 + u * 16, 16)]
                vals = nllv[pl.ds(i * 32 + u * 16, 16)]
                plsc.addupdate_scatter(repl, [lane_off + idx], vals)
            return 0

        lax.fori_loop(0, PER_TILE // 32, scat_s, 0)

        def red_s(k, _):
            base = k * 16
            acc = repl[pl.ds(base, 16)]
            for l in range(1, 16):
                acc = acc + repl[pl.ds(l * NBINP + base, 16)]
            spart[pl.ds(base, 16)] = acc
            return 0

        lax.fori_loop(0, RED_CH, red_s, 0)

        pltpu.sync_copy(cpart, shared.at[sid, 0])
        pltpu.sync_copy(spart, shared.at[sid, 1])
        plsc.subcore_barrier()

        base_slot = sid - member
        for d in range(1, NTILES):
            peer = base_slot + (member + d) % NTILES
            pltpu.sync_copy(shared.at[peer, 0], tmp)

            def addc(k, _):
                for u in range(2):
                    sl = pl.ds(k * 32 + u * 16, 16)
                    cpart[sl] = cpart[sl] + tmp[sl]
                return 0

            lax.fori_loop(0, RED_CH // 2, addc, 0)
            pltpu.sync_copy(shared.at[peer, 1], tmp)

            def adds(k, _):
                for u in range(2):
                    sl = pl.ds(k * 32 + u * 16, 16)
                    spart[sl] = spart[sl] + tmp[sl]
                return 0

            lax.fori_loop(0, RED_CH // 2, adds, 0)

        @pl.when(member == 0)
        def _():
            def fin(k, carry):
                ne_a, t_a = carry
                sl = pl.ds(k * 16, 16)
                cc = cpart[sl]
                ss = spart[sl]
                ne_a = ne_a + jnp.where(cc > 0.0, 1.0, 0.0)
                t_a = t_a + ss / jnp.maximum(cc, 1.0)
                return ne_a, t_a

            ne16, term16 = lax.fori_loop(0, RED_CH, fin, (zero16, zero16))
            term_v = zero16 + jnp.sum(term16)
            ne_v = zero16 + jnp.sum(ne16)
            outv[...] = term_v / ne_v
            pltpu.sync_copy(outv, out_hbm.at[b])

    return sc_hist


def kernel(x, target):
    x3 = x.reshape(B, C, N)
    t3 = target.reshape(B, 1, N)
    nll3, bin3 = _tc_stats(x3, t3)
    per_sample = _make_sc_kernel()(bin3.reshape(-1), nll3.reshape(-1))
    return jnp.mean(per_sample[:, 0])


# speedup vs baseline: 1.0856x; 1.0021x over previous
import functools

import jax
import jax.numpy as jnp
import numpy as np
from jax import lax
from jax.experimental import pallas as pl
from jax.experimental.pallas import tpu as pltpu
from jax.experimental.pallas import tpu_sc as plsc

B, C, H, W = 8, 96, 224, 224
N = H * W
BINS = 10
NBIN = N // BINS
NBINP = 5024
NB_BLOCKS = 2
NBLOCK = N // NB_BLOCKS
NTILES = 4
PER_TILE = N // NTILES
RED_CH = NBINP // 16


def _tc_body(x_ref, t_ref, nll_ref, bin_ref):
    xb = x_ref[0]
    t = t_ref[0]
    m = jnp.max(xb, axis=0, keepdims=True)
    s = jnp.sum(jnp.exp(xb - m), axis=0, keepdims=True)
    cls = lax.broadcasted_iota(jnp.int32, (C, NBLOCK), 0)
    xt = jnp.sum(jnp.where(cls == t, xb, 0.0), axis=0, keepdims=True)
    logp = xt - m - jnp.log(s)
    nll_ref[0] = -logp
    g = jnp.abs(jnp.exp(logp) - 1.0)
    bf = jnp.floor(g * np.float32(NBIN - 0.0001))
    bin_ref[0] = jnp.minimum(bf, NBIN - 1).astype(jnp.int32)


_tc_stats = pl.pallas_call(
    _tc_body,
    grid=(B * NB_BLOCKS,),
    in_specs=[
        pl.BlockSpec((1, C, NBLOCK), lambda i: (i // NB_BLOCKS, 0, i % NB_BLOCKS)),
        pl.BlockSpec((1, 1, NBLOCK), lambda i: (i // NB_BLOCKS, 0, i % NB_BLOCKS)),
    ],
    out_specs=[
        pl.BlockSpec((1, 1, NBLOCK), lambda i: (i // NB_BLOCKS, 0, i % NB_BLOCKS)),
        pl.BlockSpec((1, 1, NBLOCK), lambda i: (i // NB_BLOCKS, 0, i % NB_BLOCKS)),
    ],
    out_shape=[
        jax.ShapeDtypeStruct((B, 1, N), jnp.float32),
        jax.ShapeDtypeStruct((B, 1, N), jnp.int32),
    ],
)


@functools.cache
def _make_sc_kernel():
    mesh = plsc.VectorSubcoreMesh(core_axis_name="c", subcore_axis_name="s")

    @functools.partial(
        pl.kernel,
        out_type=jax.ShapeDtypeStruct((B, 16), jnp.float32),
        mesh=mesh,
        compiler_params=pltpu.CompilerParams(
            needs_layout_passes=False, use_tc_tiling_on_sc=False
        ),
        scratch_types=[
            pltpu.VMEM((16 * NBINP,), jnp.float32),
            pltpu.VMEM((PER_TILE,), jnp.int32),
            pltpu.VMEM((PER_TILE,), jnp.float32),
            pltpu.VMEM((NBINP,), jnp.float32),
            pltpu.VMEM((NBINP,), jnp.float32),
            pltpu.VMEM((NBINP,), jnp.float32),
            pltpu.VMEM((16,), jnp.float32),
            pltpu.VMEM_SHARED((16, 2, NBINP), jnp.float32),
        ],
    )
    def sc_hist(bins_hbm, nll_hbm, out_hbm, repl, binsv, nllv, cpart, spart,
                tmp, outv, shared):
        cid = lax.axis_index("c")
        sid = lax.axis_index("s")
        b = cid * (16 // NTILES) + sid // NTILES
        member = sid % NTILES
        off = b * N + member * PER_TILE

        zero16 = jnp.zeros((16,), jnp.float32)
        ones16 = jnp.full((16,), 1.0, jnp.float32)
        lane = lax.broadcasted_iota(jnp.int32, (16,), 0)

        def zero_body(j, _):
            for u in range(8):
                repl[pl.ds(j * 128 + u * 16, 16)] = zero16
            return 0

        lax.fori_loop(0, 16 * NBINP // 128, zero_body, 0)

        pltpu.sync_copy(bins_hbm.at[pl.ds(off, PER_TILE)], binsv)
        pltpu.sync_copy(nll_hbm.at[pl.ds(off, PER_TILE)], nllv)

        lane_off = lane * NBINP

        def scat_c(i, _):
            for u in range(2):
                idx = binsv[pl.ds(i * 32 + u * 16, 16)]
                plsc.addupdate_scatter(repl, [lane_off + idx], ones16)
            return 0

        lax.fori_loop(0, PER_TILE // 32, scat_c, 0)

        def red_c(k, _):
            for u in range(2):
                base = k * 32 + u * 16
                acc = repl[pl.ds(base, 16)]
                repl[pl.ds(base, 16)] = zero16
                for l in range(1, 16):
                    o = l * NBINP + base
                    acc = acc + repl[pl.ds(o, 16)]
                    repl[pl.ds(o, 16)] = zero16
                cpart[pl.ds(base, 16)] = acc
            return 0

        lax.fori_loop(0, RED_CH // 2, red_c, 0)

        def scat_s(i, _):
            for u in range(2):
                idx = binsv[pl.ds(i * 32 + u * 16, 16)]
                vals = nllv[pl.ds(i * 32 + u * 16, 16)]
                plsc.addupdate_scatter(repl, [lane_off + idx], vals)
            return 0

        lax.fori_loop(0, PER_TILE // 32, scat_s, 0)

        def red_s(k, _):
            for u in range(2):
                base = k * 32 + u * 16
                acc = repl[pl.ds(base, 16)]
                for l in range(1, 16):
                    acc = acc + repl[pl.ds(l * NBINP + base, 16)]
                spart[pl.ds(base, 16)] = acc
            return 0

        lax.fori_loop(0, RED_CH // 2, red_s, 0)

        pltpu.sync_copy(cpart, shared.at[sid, 0])
        pltpu.sync_copy(spart, shared.at[sid, 1])
        plsc.subcore_barrier()

        base_slot = sid - member
        for d in range(1, NTILES):
            peer = base_slot + (member + d) % NTILES
            pltpu.sync_copy(shared.at[peer, 0], tmp)

            def addc(k, _):
                for u in range(2):
                    sl = pl.ds(k * 32 + u * 16, 16)
                    cpart[sl] = cpart[sl] + tmp[sl]
                return 0

            lax.fori_loop(0, RED_CH // 2, addc, 0)
            pltpu.sync_copy(shared.at[peer, 1], tmp)

            def adds(k, _):
                for u in range(2):
                    sl = pl.ds(k * 32 + u * 16, 16)
                    spart[sl] = spart[sl] + tmp[sl]
                return 0

            lax.fori_loop(0, RED_CH // 2, adds, 0)

        @pl.when(member == 0)
        def _():
            def fin(k, carry):
                ne_a, t_a = carry
                for u in range(2):
                    sl = pl.ds(k * 32 + u * 16, 16)
                    cc = cpart[sl]
                    ss = spart[sl]
                    ne_a = ne_a + jnp.where(cc > 0.0, 1.0, 0.0)
                    t_a = t_a + ss / jnp.maximum(cc, 1.0)
                return ne_a, t_a

            ne16, term16 = lax.fori_loop(0, RED_CH // 2, fin, (zero16, zero16))
            term_v = zero16 + jnp.sum(term16)
            ne_v = zero16 + jnp.sum(ne16)
            outv[...] = term_v / ne_v
            pltpu.sync_copy(outv, out_hbm.at[b])

    return sc_hist


def kernel(x, target):
    x3 = x.reshape(B, C, N)
    t3 = target.reshape(B, 1, N)
    nll3, bin3 = _tc_stats(x3, t3)
    per_sample = _make_sc_kernel()(bin3.reshape(-1), nll3.reshape(-1))
    return jnp.mean(per_sample[:, 0])
